# Initial kernel scaffold; baseline (speedup 1.0000x reference)
import jax
import jax.numpy as jnp
from jax.experimental import pallas as pl


def kernel(weight, edge_index, idx):
    raise NotImplementedError("write your pallas kernel here")


# speedup vs baseline: 9.6151x; 9.6151x over previous
import functools

import jax
import jax.numpy as jnp
from jax import lax
from jax.experimental import pallas as pl
from jax.experimental.pallas import tpu as pltpu
from jax.experimental.pallas import tpu_sc as plsc

N_NODES = 10000
DIM = 128
C = 1.0
EPS = 1e-7
N_EDGES = 320000
B_DECODE = 8192

NC = 2
NS = 16
NW = NC * NS
CH = 128

EPW = N_EDGES // NW
NCHUNK = -(-EPW // CH)
PADW = NCHUNK * CH - EPW

DUMMY = N_NODES
NACC = 10368
RPT = NACC // NS

_MESH = plsc.VectorSubcoreMesh(
    core_axis_name="c", subcore_axis_name="s", num_cores=NC, num_subcores=NS)


@functools.partial(
    pl.kernel,
    out_type=jax.ShapeDtypeStruct((NC, NACC, DIM), jnp.float32),
    mesh=_MESH,
    scratch_types=[
        pltpu.VMEM((NCHUNK, CH), jnp.int32),
        pltpu.VMEM((CH, DIM), jnp.float32),
        pltpu.VMEM_SHARED((NACC, DIM), jnp.float32),
    ],
)
def _deg_kernel(dst_hbm, out_hbm, idst, buf, acc):
    c = lax.axis_index("c")
    s = lax.axis_index("s")
    w = s * NC + c
    pltpu.sync_copy(dst_hbm.at[w], idst)

    def fill(val):
        def body(r, _):
            for k in range(DIM // 16):
                buf[r, pl.ds(k * 16, 16)] = val
            return 0
        lax.fori_loop(0, CH, body, 0)

    fill(jnp.zeros((16,), jnp.float32))
    base = s * RPT
    for k in range(RPT // CH):
        pltpu.sync_copy(buf, acc.at[pl.ds(base + k * CH, CH)])
    pltpu.sync_copy(buf.at[pl.ds(0, 8)],
                    acc.at[pl.ds(base + (RPT // CH) * CH, 8)])
    fill(jnp.ones((16,), jnp.float32))
    plsc.subcore_barrier()

    def scat_body(j, _):
        pltpu.sync_copy(buf, acc.at[idst.at[j]], add=True)
        return 0
    lax.fori_loop(0, NCHUNK, scat_body, 0)
    plsc.subcore_barrier()

    for k in range(RPT // CH):
        r0 = base + k * CH
        pltpu.sync_copy(acc.at[pl.ds(r0, CH)], buf)
        pltpu.sync_copy(buf, out_hbm.at[c, pl.ds(r0, CH)])
    r1 = base + (RPT // CH) * CH
    pltpu.sync_copy(acc.at[pl.ds(r1, 8)], buf.at[pl.ds(0, 8)])
    pltpu.sync_copy(buf.at[pl.ds(0, 8)], out_hbm.at[c, pl.ds(r1, 8)])


@functools.partial(
    pl.kernel,
    out_type=jax.ShapeDtypeStruct((NC, NACC, DIM), jnp.float32),
    mesh=_MESH,
    scratch_types=[
        pltpu.VMEM((NCHUNK, CH), jnp.int32),
        pltpu.VMEM((NCHUNK, CH), jnp.int32),
        pltpu.VMEM((CH, DIM), jnp.float32),
        pltpu.VMEM_SHARED((NACC, DIM), jnp.float32),
        pltpu.SemaphoreType.DMA,
    ],
)
def _edge_kernel(u_hbm, src_hbm, dst_hbm, out_hbm,
                 isrc, idst, rows, acc, sem):
    c = lax.axis_index("c")
    s = lax.axis_index("s")
    w = s * NC + c
    pltpu.sync_copy(src_hbm.at[w], isrc)
    pltpu.sync_copy(dst_hbm.at[w], idst)

    zeros_v = jnp.zeros((16,), jnp.float32)

    def zero_body(r, _):
        for k in range(DIM // 16):
            rows[r, pl.ds(k * 16, 16)] = zeros_v
        return 0
    lax.fori_loop(0, CH, zero_body, 0)
    base = s * RPT
    for k in range(RPT // CH):
        pltpu.sync_copy(rows, acc.at[pl.ds(base + k * CH, CH)])
    pltpu.sync_copy(rows.at[pl.ds(0, 8)],
                    acc.at[pl.ds(base + (RPT // CH) * CH, 8)])
    plsc.subcore_barrier()

    def edge_body(j, _):
        pltpu.async_copy(u_hbm.at[isrc.at[j]], rows, sem).wait()
        pltpu.sync_copy(rows, acc.at[idst.at[j]], add=True)
        return 0
    lax.fori_loop(0, NCHUNK, edge_body, 0)
    plsc.subcore_barrier()

    for k in range(RPT // CH):
        r0 = base + k * CH
        pltpu.sync_copy(acc.at[pl.ds(r0, CH)], rows)
        pltpu.sync_copy(rows, out_hbm.at[c, pl.ds(r0, CH)])
    r1 = base + (RPT // CH) * CH
    pltpu.sync_copy(acc.at[pl.ds(r1, 8)], rows.at[pl.ds(0, 8)])
    pltpu.sync_copy(rows.at[pl.ds(0, 8)], out_hbm.at[c, pl.ds(r1, 8)])


@functools.partial(
    pl.kernel,
    out_type=(jax.ShapeDtypeStruct((B_DECODE, DIM), jnp.float32),
              jax.ShapeDtypeStruct((B_DECODE, DIM), jnp.float32)),
    mesh=_MESH,
    scratch_types=[
        pltpu.VMEM((2, 128), jnp.int32),
        pltpu.VMEM((2, 128), jnp.int32),
        pltpu.VMEM((128, DIM), jnp.float32),
        pltpu.SemaphoreType.DMA,
    ],
)
def _decode_gather(h_hbm, i0_hbm, i1_hbm, e0_hbm, e1_hbm,
                   ib0, ib1, rows, sem):
    c = lax.axis_index("c")
    s = lax.axis_index("s")
    w = s * NC + c
    pltpu.sync_copy(i0_hbm.at[w], ib0)
    pltpu.sync_copy(i1_hbm.at[w], ib1)
    for j in range(2):
        base = w * 256 + j * 128
        pltpu.async_copy(h_hbm.at[ib0.at[j]], rows, sem).wait()
        pltpu.sync_copy(rows, e0_hbm.at[pl.ds(base, 128)])
        pltpu.async_copy(h_hbm.at[ib1.at[j]], rows, sem).wait()
        pltpu.sync_copy(rows, e1_hbm.at[pl.ds(base, 128)])


_RB = 1000


def _lane_mask(shape):
    return lax.broadcasted_iota(jnp.int32, shape, len(shape) - 1) == 0


def _arccosh(x):
    return jnp.log(x + jnp.sqrt(x * x - 1.0))


def _expmap(v):
    m0 = _lane_mask(v.shape)
    vm = jnp.where(m0, 0.0, v)
    vn = jnp.sqrt(jnp.clip(jnp.sum(vm * vm, axis=-1, keepdims=True), EPS, None))
    e = jnp.exp(vn)
    ei = 1.0 / e
    cosh = 0.5 * (e + ei)
    sinh = 0.5 * (e - ei)
    return jnp.where(m0, cosh, sinh / vn * vm)


def _logmap(x):
    m0 = _lane_mask(x.shape)
    x0 = jnp.clip(jnp.sum(jnp.where(m0, x, 0.0), axis=-1, keepdims=True),
                  1.0 + EPS, None)
    d = _arccosh(x0)
    rest = jnp.where(m0, 0.0, x)
    rn = jnp.sqrt(jnp.clip(jnp.sum(rest * rest, axis=-1, keepdims=True),
                           EPS, None))
    return jnp.where(m0, 0.0, d / rn * rest)


def _invd(degp):
    deg = degp[0, :, 0:1] + degp[1, :, 0:1]
    return lax.rsqrt(jnp.maximum(deg, 1.0))


def _dense1_body(w_ref, degp_ref, u_ref):
    invd = _invd(degp_ref[...])
    u_ref[...] = _logmap(_expmap(w_ref[...])) * invd


def _dense2_body(aggp_ref, degp_ref, u_ref):
    invd = _invd(degp_ref[...])
    agg = (aggp_ref[0] + aggp_ref[1]) * invd
    u_ref[...] = _logmap(_expmap(agg)) * invd


def _dense3_body(aggp_ref, degp_ref, h_ref):
    invd = _invd(degp_ref[...])
    h_ref[...] = _expmap((aggp_ref[0] + aggp_ref[1]) * invd)


_tab_spec = pl.BlockSpec((_RB, DIM), lambda i: (i, 0))
_degp_spec = pl.BlockSpec((NC, _RB, DIM), lambda i: (0, i, 0))
_aggp_spec = pl.BlockSpec((NC, _RB, DIM), lambda i: (0, i, 0))

_dense1 = pl.pallas_call(
    _dense1_body,
    grid=(N_NODES // _RB,),
    in_specs=[_tab_spec, _degp_spec],
    out_specs=_tab_spec,
    out_shape=jax.ShapeDtypeStruct((N_NODES, DIM), jnp.float32),
)

_dense2 = pl.pallas_call(
    _dense2_body,
    grid=(N_NODES // _RB,),
    in_specs=[_aggp_spec, _degp_spec],
    out_specs=_tab_spec,
    out_shape=jax.ShapeDtypeStruct((N_NODES, DIM), jnp.float32),
)

_dense3 = pl.pallas_call(
    _dense3_body,
    grid=(N_NODES // _RB,),
    in_specs=[_aggp_spec, _degp_spec],
    out_specs=_tab_spec,
    out_shape=jax.ShapeDtypeStruct((N_NODES, DIM), jnp.float32),
)


def _dmath_body(e0_ref, e1_ref, o_ref):
    p = e0_ref[...] * e1_ref[...]
    m0 = _lane_mask(p.shape)
    s = jnp.sum(p, axis=-1, keepdims=True)
    c0 = jnp.sum(jnp.where(m0, p, 0.0), axis=-1, keepdims=True)
    prod = s - 2.0 * c0
    theta = jnp.clip(-prod * C, 1.0 + EPS, None)
    sq = (1.0 / C) * _arccosh(theta) ** 2
    o_ref[...] = jnp.clip(sq, None, 50.0)


_DB = 1024
_dmath = pl.pallas_call(
    _dmath_body,
    grid=(B_DECODE // _DB,),
    in_specs=[pl.BlockSpec((_DB, DIM), lambda i: (i, 0)),
              pl.BlockSpec((_DB, DIM), lambda i: (i, 0))],
    out_specs=pl.BlockSpec((_DB, 1), lambda i: (i, 0)),
    out_shape=jax.ShapeDtypeStruct((B_DECODE, 1), jnp.float32),
)


def kernel(weight, edge_index, idx):
    padw = ((0, 0), (0, PADW))
    src = jnp.pad(edge_index[0].reshape(NW, EPW), padw,
                  constant_values=0).reshape(NW, NCHUNK, CH)
    dst = jnp.pad(edge_index[1].reshape(NW, EPW), padw,
                  constant_values=DUMMY).reshape(NW, NCHUNK, CH)
    degp = _deg_kernel(dst)[:, :N_NODES, :]
    u1 = _dense1(weight, degp)
    agg1p = _edge_kernel(u1, src, dst)[:, :N_NODES, :]
    u2 = _dense2(agg1p, degp)
    agg2p = _edge_kernel(u2, src, dst)[:, :N_NODES, :]
    h = _dense3(agg2p, degp)
    i0 = idx[:, 0].reshape(NW, 2, 128)
    i1 = idx[:, 1].reshape(NW, 2, 128)
    e0, e1 = _decode_gather(h, i0, i1)
    return _dmath(e0, e1)


# speedup vs baseline: 10.8133x; 1.1246x over previous
import functools

import jax
import jax.numpy as jnp
from jax import lax
from jax.experimental import pallas as pl
from jax.experimental.pallas import tpu as pltpu
from jax.experimental.pallas import tpu_sc as plsc

N_NODES = 10000
DIM = 128
C = 1.0
EPS = 1e-7
N_EDGES = 320000
B_DECODE = 8192

NC = 2
NS = 16
NW = NC * NS
CH = 128

EPW = N_EDGES // NW
NCHUNK = -(-EPW // CH)
PADW = NCHUNK * CH - EPW

DUMMY = N_NODES
NACC = 10368
RPT = NACC // NS

_MESH = plsc.VectorSubcoreMesh(
    core_axis_name="c", subcore_axis_name="s", num_cores=NC, num_subcores=NS)


@functools.partial(
    pl.kernel,
    out_type=jax.ShapeDtypeStruct((NC, NACC, DIM), jnp.float32),
    mesh=_MESH,
    scratch_types=[
        pltpu.VMEM((NCHUNK, CH), jnp.int32),
        pltpu.VMEM((CH, DIM), jnp.float32),
        pltpu.VMEM_SHARED((NACC, DIM), jnp.float32),
    ],
)
def _deg_kernel(dst_hbm, out_hbm, idst, buf, acc):
    c = lax.axis_index("c")
    s = lax.axis_index("s")
    w = s * NC + c
    pltpu.sync_copy(dst_hbm.at[w], idst)

    def fill(val):
        def body(r, _):
            for k in range(DIM // 16):
                buf[r, pl.ds(k * 16, 16)] = val
            return 0
        lax.fori_loop(0, CH, body, 0)

    fill(jnp.zeros((16,), jnp.float32))
    base = s * RPT
    for k in range(RPT // CH):
        pltpu.sync_copy(buf, acc.at[pl.ds(base + k * CH, CH)])
    pltpu.sync_copy(buf.at[pl.ds(0, 8)],
                    acc.at[pl.ds(base + (RPT // CH) * CH, 8)])
    fill(jnp.ones((16,), jnp.float32))
    plsc.subcore_barrier()

    def scat_body(j, _):
        pltpu.sync_copy(buf, acc.at[idst.at[j]], add=True)
        return 0
    lax.fori_loop(0, NCHUNK, scat_body, 0)
    plsc.subcore_barrier()

    for k in range(RPT // CH):
        r0 = base + k * CH
        pltpu.sync_copy(acc.at[pl.ds(r0, CH)], buf)
        pltpu.sync_copy(buf, out_hbm.at[c, pl.ds(r0, CH)])
    r1 = base + (RPT // CH) * CH
    pltpu.sync_copy(acc.at[pl.ds(r1, 8)], buf.at[pl.ds(0, 8)])
    pltpu.sync_copy(buf.at[pl.ds(0, 8)], out_hbm.at[c, pl.ds(r1, 8)])


@functools.partial(
    pl.kernel,
    out_type=jax.ShapeDtypeStruct((NC, NACC, DIM), jnp.float32),
    mesh=_MESH,
    scratch_types=[
        pltpu.VMEM((NCHUNK, CH), jnp.int32),
        pltpu.VMEM((2, CH), jnp.int32),
        pltpu.VMEM((2, CH, DIM), jnp.float32),
        pltpu.VMEM_SHARED((NACC, DIM), jnp.float32),
        pltpu.SemaphoreType.DMA((2,)),
        pltpu.SemaphoreType.DMA((2,)),
        pltpu.SemaphoreType.DMA((2,)),
    ],
)
def _edge_kernel(u_hbm, src_hbm, dst_hbm, out_hbm,
                 isrc, idst2, rows2, acc, gs, ss, ds):
    c = lax.axis_index("c")
    s = lax.axis_index("s")
    w = s * NC + c
    pltpu.sync_copy(src_hbm.at[w], isrc)

    zeros_v = jnp.zeros((16,), jnp.float32)

    def zero_body(r, _):
        for k in range(DIM // 16):
            rows2[0, r, pl.ds(k * 16, 16)] = zeros_v
        return 0
    lax.fori_loop(0, CH, zero_body, 0)
    base = s * RPT
    for k in range(RPT // CH):
        pltpu.sync_copy(rows2.at[0], acc.at[pl.ds(base + k * CH, CH)])
    pltpu.sync_copy(rows2.at[0].at[pl.ds(0, 8)],
                    acc.at[pl.ds(base + (RPT // CH) * CH, 8)])
    plsc.subcore_barrier()

    def start_gather(j, b):
        pltpu.async_copy(u_hbm.at[isrc.at[j]], rows2.at[b], gs.at[b])

    def start_dst(j, b):
        pltpu.async_copy(dst_hbm.at[w, j], idst2.at[b], ds.at[b])

    start_gather(0, 0)
    start_dst(0, 0)

    def edge_body(j, _):
        b = j & 1
        pltpu.make_async_copy(dst_hbm.at[w, j], idst2.at[b], ds.at[b]).wait()
        pltpu.make_async_copy(u_hbm.at[isrc.at[j]], rows2.at[b],
                              gs.at[b]).wait()
        pltpu.async_copy(rows2.at[b], acc.at[idst2.at[b]], ss.at[b], add=True)

        @pl.when(j >= 1)
        def _():
            pltpu.make_async_copy(rows2.at[1 - b], acc.at[idst2.at[1 - b]],
                                  ss.at[1 - b]).wait()

        @pl.when(j + 1 < NCHUNK)
        def _():
            start_gather(j + 1, 1 - b)
            start_dst(j + 1, 1 - b)
        return 0
    lax.fori_loop(0, NCHUNK, edge_body, 0)
    bl = (NCHUNK - 1) & 1
    pltpu.make_async_copy(rows2.at[bl], acc.at[idst2.at[bl]],
                          ss.at[bl]).wait()
    plsc.subcore_barrier()

    for k in range(RPT // CH):
        r0 = base + k * CH
        pltpu.sync_copy(acc.at[pl.ds(r0, CH)], rows2.at[0])
        pltpu.sync_copy(rows2.at[0], out_hbm.at[c, pl.ds(r0, CH)])
    r1 = base + (RPT // CH) * CH
    pltpu.sync_copy(acc.at[pl.ds(r1, 8)], rows2.at[0].at[pl.ds(0, 8)])
    pltpu.sync_copy(rows2.at[0].at[pl.ds(0, 8)], out_hbm.at[c, pl.ds(r1, 8)])


@functools.partial(
    pl.kernel,
    out_type=(jax.ShapeDtypeStruct((B_DECODE, DIM), jnp.float32),
              jax.ShapeDtypeStruct((B_DECODE, DIM), jnp.float32)),
    mesh=_MESH,
    scratch_types=[
        pltpu.VMEM((2, 128), jnp.int32),
        pltpu.VMEM((2, 128), jnp.int32),
        pltpu.VMEM((128, DIM), jnp.float32),
        pltpu.SemaphoreType.DMA,
    ],
)
def _decode_gather(h_hbm, i0_hbm, i1_hbm, e0_hbm, e1_hbm,
                   ib0, ib1, rows, sem):
    c = lax.axis_index("c")
    s = lax.axis_index("s")
    w = s * NC + c
    pltpu.sync_copy(i0_hbm.at[w], ib0)
    pltpu.sync_copy(i1_hbm.at[w], ib1)
    for j in range(2):
        base = w * 256 + j * 128
        pltpu.async_copy(h_hbm.at[ib0.at[j]], rows, sem).wait()
        pltpu.sync_copy(rows, e0_hbm.at[pl.ds(base, 128)])
        pltpu.async_copy(h_hbm.at[ib1.at[j]], rows, sem).wait()
        pltpu.sync_copy(rows, e1_hbm.at[pl.ds(base, 128)])


_RB = 1000


def _lane_mask(shape):
    return lax.broadcasted_iota(jnp.int32, shape, len(shape) - 1) == 0


def _arccosh(x):
    return jnp.log(x + jnp.sqrt(x * x - 1.0))


def _expmap(v):
    m0 = _lane_mask(v.shape)
    vm = jnp.where(m0, 0.0, v)
    vn = jnp.sqrt(jnp.clip(jnp.sum(vm * vm, axis=-1, keepdims=True), EPS, None))
    e = jnp.exp(vn)
    ei = 1.0 / e
    cosh = 0.5 * (e + ei)
    sinh = 0.5 * (e - ei)
    return jnp.where(m0, cosh, sinh / vn * vm)


def _logmap(x):
    m0 = _lane_mask(x.shape)
    x0 = jnp.clip(jnp.sum(jnp.where(m0, x, 0.0), axis=-1, keepdims=True),
                  1.0 + EPS, None)
    d = _arccosh(x0)
    rest = jnp.where(m0, 0.0, x)
    rn = jnp.sqrt(jnp.clip(jnp.sum(rest * rest, axis=-1, keepdims=True),
                           EPS, None))
    return jnp.where(m0, 0.0, d / rn * rest)


def _invd(degp):
    deg = degp[0, :, 0:1] + degp[1, :, 0:1]
    return lax.rsqrt(jnp.maximum(deg, 1.0))


def _dense1_body(w_ref, degp_ref, u_ref):
    invd = _invd(degp_ref[...])
    u_ref[...] = _logmap(_expmap(w_ref[...])) * invd


def _dense2_body(aggp_ref, degp_ref, u_ref):
    invd = _invd(degp_ref[...])
    agg = (aggp_ref[0] + aggp_ref[1]) * invd
    u_ref[...] = _logmap(_expmap(agg)) * invd


def _dense3_body(aggp_ref, degp_ref, h_ref):
    invd = _invd(degp_ref[...])
    h_ref[...] = _expmap((aggp_ref[0] + aggp_ref[1]) * invd)


_tab_spec = pl.BlockSpec((_RB, DIM), lambda i: (i, 0))
_degp_spec = pl.BlockSpec((NC, _RB, DIM), lambda i: (0, i, 0))
_aggp_spec = pl.BlockSpec((NC, _RB, DIM), lambda i: (0, i, 0))

_dense1 = pl.pallas_call(
    _dense1_body,
    grid=(N_NODES // _RB,),
    in_specs=[_tab_spec, _degp_spec],
    out_specs=_tab_spec,
    out_shape=jax.ShapeDtypeStruct((N_NODES, DIM), jnp.float32),
)

_dense2 = pl.pallas_call(
    _dense2_body,
    grid=(N_NODES // _RB,),
    in_specs=[_aggp_spec, _degp_spec],
    out_specs=_tab_spec,
    out_shape=jax.ShapeDtypeStruct((N_NODES, DIM), jnp.float32),
)

_dense3 = pl.pallas_call(
    _dense3_body,
    grid=(N_NODES // _RB,),
    in_specs=[_aggp_spec, _degp_spec],
    out_specs=_tab_spec,
    out_shape=jax.ShapeDtypeStruct((N_NODES, DIM), jnp.float32),
)


def _dmath_body(e0_ref, e1_ref, o_ref):
    p = e0_ref[...] * e1_ref[...]
    m0 = _lane_mask(p.shape)
    s = jnp.sum(p, axis=-1, keepdims=True)
    c0 = jnp.sum(jnp.where(m0, p, 0.0), axis=-1, keepdims=True)
    prod = s - 2.0 * c0
    theta = jnp.clip(-prod * C, 1.0 + EPS, None)
    sq = (1.0 / C) * _arccosh(theta) ** 2
    o_ref[...] = jnp.clip(sq, None, 50.0)


_DB = 1024
_dmath = pl.pallas_call(
    _dmath_body,
    grid=(B_DECODE // _DB,),
    in_specs=[pl.BlockSpec((_DB, DIM), lambda i: (i, 0)),
              pl.BlockSpec((_DB, DIM), lambda i: (i, 0))],
    out_specs=pl.BlockSpec((_DB, 1), lambda i: (i, 0)),
    out_shape=jax.ShapeDtypeStruct((B_DECODE, 1), jnp.float32),
)


def kernel(weight, edge_index, idx):
    padw = ((0, 0), (0, PADW))
    src = jnp.pad(edge_index[0].reshape(NW, EPW), padw,
                  constant_values=0).reshape(NW, NCHUNK, CH)
    dst = jnp.pad(edge_index[1].reshape(NW, EPW), padw,
                  constant_values=DUMMY).reshape(NW, NCHUNK, CH)
    degp = _deg_kernel(dst)[:, :N_NODES, :]
    u1 = _dense1(weight, degp)
    agg1p = _edge_kernel(u1, src, dst)[:, :N_NODES, :]
    u2 = _dense2(agg1p, degp)
    agg2p = _edge_kernel(u2, src, dst)[:, :N_NODES, :]
    h = _dense3(agg2p, degp)
    i0 = idx[:, 0].reshape(NW, 2, 128)
    i1 = idx[:, 1].reshape(NW, 2, 128)
    e0, e1 = _decode_gather(h, i0, i1)
    return _dmath(e0, e1)
